# Initial kernel scaffold; baseline (speedup 1.0000x reference)
import jax
import jax.numpy as jnp
from jax.experimental import pallas as pl


def kernel(x, edge_index0, edge_index1, W0, W1, b):
    raise NotImplementedError("write your pallas kernel here")


# speedup vs baseline: 4.7841x; 4.7841x over previous
import functools

import jax
import jax.numpy as jnp
from jax import lax
from jax.experimental import pallas as pl
from jax.experimental.pallas import tpu as pltpu
from jax.experimental.pallas import tpu_sc as plsc

N = 10000
D = 128
E = 320000
NC = 2
NS = 16
EPW = E // (NC * NS)
BLK = 80
NB = EPW // BLK
RPT = 624
TAIL = N - RPT * NS
TAIL0 = RPT * NS


def _mm_body(x_ref, w0_ref, w1_ref, o0_ref, o1_ref):
    xb = x_ref[...]
    o0_ref[...] = jnp.dot(xb, w0_ref[...], preferred_element_type=jnp.float32)
    o1_ref[...] = jnp.dot(xb, w1_ref[...], preferred_element_type=jnp.float32)


def _combine_body(p0_ref, p1_ref, b_ref, o_ref):
    o_ref[...] = jnp.maximum(p0_ref[...] + p1_ref[...] + b_ref[...], 0.0)


def _sc_segment_sum(pre0, pre1, src0, dst0, src1, dst1, zeros):
    mesh = plsc.VectorSubcoreMesh(core_axis_name="c", subcore_axis_name="s")

    @functools.partial(
        pl.kernel,
        mesh=mesh,
        out_type=[jax.ShapeDtypeStruct((N, D), jnp.float32)] * 2,
        scratch_types=[
            pltpu.VMEM_SHARED((N, D), jnp.float32),
            pltpu.VMEM((BLK,), jnp.int32),
            pltpu.VMEM((BLK,), jnp.int32),
            pltpu.VMEM((BLK, D), jnp.float32),
            pltpu.SemaphoreType.DMA,
        ],
    )
    def k(pre0_hbm, pre1_hbm, src0_hbm, dst0_hbm, src1_hbm, dst1_hbm, z_hbm,
          out0_hbm, out1_hbm, acc, idx_s, idx_d, rows, sem):
        c = lax.axis_index("c")
        s = lax.axis_index("s")
        wid = c * NS + s
        row0 = s * RPT
        pltpu.sync_copy(z_hbm.at[pl.ds(row0, RPT)], acc.at[pl.ds(row0, RPT)])

        @pl.when(s == NS - 1)
        def _():
            pltpu.sync_copy(z_hbm.at[pl.ds(TAIL0, TAIL)],
                            acc.at[pl.ds(TAIL0, TAIL)])

        plsc.subcore_barrier()
        base = wid * EPW
        for pre_hbm, src_hbm, dst_hbm in ((pre0_hbm, src0_hbm, dst0_hbm),
                                          (pre1_hbm, src1_hbm, dst1_hbm)):
            def body(j, carry):
                off = base + j * BLK
                pltpu.sync_copy(src_hbm.at[pl.ds(off, BLK)], idx_s)
                pltpu.sync_copy(dst_hbm.at[pl.ds(off, BLK)], idx_d)
                pltpu.async_copy(pre_hbm.at[idx_s], rows, sem).wait()
                pltpu.sync_copy(rows, acc.at[idx_d], add=True)
                return carry

            lax.fori_loop(0, NB, body, 0)
        plsc.subcore_barrier()

        @pl.when(c == 0)
        def _():
            pltpu.sync_copy(acc.at[pl.ds(row0, RPT)], out0_hbm.at[pl.ds(row0, RPT)])

            @pl.when(s == NS - 1)
            def _():
                pltpu.sync_copy(acc.at[pl.ds(TAIL0, TAIL)],
                                out0_hbm.at[pl.ds(TAIL0, TAIL)])

        @pl.when(c == 1)
        def _():
            pltpu.sync_copy(acc.at[pl.ds(row0, RPT)], out1_hbm.at[pl.ds(row0, RPT)])

            @pl.when(s == NS - 1)
            def _():
                pltpu.sync_copy(acc.at[pl.ds(TAIL0, TAIL)],
                                out1_hbm.at[pl.ds(TAIL0, TAIL)])

    return k(pre0, pre1, src0, dst0, src1, dst1, zeros)


def kernel(x, edge_index0, edge_index1, W0, W1, b):
    src0 = edge_index0[0].astype(jnp.int32)
    dst0 = edge_index0[1].astype(jnp.int32)
    src1 = edge_index1[0].astype(jnp.int32)
    dst1 = edge_index1[1].astype(jnp.int32)
    pre0, pre1 = pl.pallas_call(
        _mm_body,
        grid=(5,),
        in_specs=[pl.BlockSpec((2000, D), lambda i: (i, 0)),
                  pl.BlockSpec((D, D), lambda i: (0, 0)),
                  pl.BlockSpec((D, D), lambda i: (0, 0))],
        out_specs=[pl.BlockSpec((2000, D), lambda i: (i, 0)),
                   pl.BlockSpec((2000, D), lambda i: (i, 0))],
        out_shape=[jax.ShapeDtypeStruct((N, D), jnp.float32)] * 2,
    )(x, W0, W1)
    zeros = jnp.zeros((N, D), jnp.float32)
    part0, part1 = _sc_segment_sum(pre0, pre1, src0, dst0, src1, dst1, zeros)
    b2 = jnp.reshape(b, (1, D))
    out = pl.pallas_call(
        _combine_body,
        grid=(5,),
        in_specs=[pl.BlockSpec((2000, D), lambda i: (i, 0)),
                  pl.BlockSpec((2000, D), lambda i: (i, 0)),
                  pl.BlockSpec((1, D), lambda i: (0, 0))],
        out_specs=pl.BlockSpec((2000, D), lambda i: (i, 0)),
        out_shape=jax.ShapeDtypeStruct((N, D), jnp.float32),
    )(part0, part1, b2)
    return out


# speedup vs baseline: 10.9584x; 2.2906x over previous
import functools

import jax
import jax.numpy as jnp
from jax import lax
from jax.experimental import pallas as pl
from jax.experimental.pallas import tpu as pltpu
from jax.experimental.pallas import tpu_sc as plsc

N = 10000
D = 128
E = 320000
NC = 2
NS = 16
NW = NC * NS
EPW = 2 * E // NW
BLK = 80
NBW = EPW // BLK
RPT = 624
TAIL = N - RPT * NS
TAIL0 = RPT * NS


def _mm_body(x_ref, w_ref, o_ref):
    o_ref[...] = jnp.dot(x_ref[...], w_ref[0],
                         preferred_element_type=jnp.float32)


def _combine_body(p0_ref, p1_ref, b_ref, o_ref):
    o_ref[...] = jnp.maximum(p0_ref[...] + p1_ref[...] + b_ref[...], 0.0)


def _sc_segment_sum(pre, srcc, dstc, zeros):
    mesh = plsc.VectorSubcoreMesh(core_axis_name="c", subcore_axis_name="s")

    @functools.partial(
        pl.kernel,
        mesh=mesh,
        out_type=[jax.ShapeDtypeStruct((N, D), jnp.float32)] * 2,
        scratch_types=[
            pltpu.VMEM_SHARED((N, D), jnp.float32),
            pltpu.VMEM((4, BLK), jnp.int32),
            pltpu.VMEM((4, BLK), jnp.int32),
            pltpu.VMEM((BLK, D), jnp.float32),
            pltpu.VMEM((BLK, D), jnp.float32),
            pltpu.SemaphoreType.DMA,
            pltpu.SemaphoreType.DMA,
            pltpu.SemaphoreType.DMA,
            pltpu.SemaphoreType.DMA,
            pltpu.SemaphoreType.DMA,
            pltpu.SemaphoreType.DMA,
        ],
    )
    def k(pre_hbm, src_hbm, dst_hbm, z_hbm, out0_hbm, out1_hbm,
          acc, idx_s, idx_d, rows0, rows1, si0, si1, si2, si3, sg0, sg1):
        c = lax.axis_index("c")
        s = lax.axis_index("s")
        wid = c * NS + s
        row0 = s * RPT
        base = wid * EPW
        sis = (si0, si1, si2, si3)
        rowbufs = (rows0, rows1)
        sgs = (sg0, sg1)

        def start_idx(j, u, sem):
            pltpu.async_copy(src_hbm.at[pl.ds(base + j * BLK, BLK)],
                             idx_s.at[u], sem)
            pltpu.async_copy(dst_hbm.at[pl.ds(base + j * BLK, BLK)],
                             idx_d.at[u], sem)

        def wait_idx(u, sem):
            pltpu.make_async_copy(src_hbm.at[pl.ds(0, BLK)],
                                  idx_s.at[u], sem).wait()
            pltpu.make_async_copy(dst_hbm.at[pl.ds(0, BLK)],
                                  idx_d.at[u], sem).wait()

        pltpu.sync_copy(z_hbm.at[pl.ds(row0, RPT)], acc.at[pl.ds(row0, RPT)])

        @pl.when(s == NS - 1)
        def _():
            pltpu.sync_copy(z_hbm.at[pl.ds(TAIL0, TAIL)],
                            acc.at[pl.ds(TAIL0, TAIL)])

        plsc.subcore_barrier()

        for u in range(4):
            start_idx(u, u, sis[u])
        for u in range(2):
            wait_idx(u, sis[u])
            pltpu.async_copy(pre_hbm.at[idx_s.at[u]], rowbufs[u], sgs[u])

        def sub(j, u):
            r = rowbufs[u % 2]
            sg = sgs[u % 2]
            pltpu.make_async_copy(pre_hbm.at[idx_s.at[u]], r, sg).wait()
            pltpu.sync_copy(r, acc.at[idx_d.at[u]], add=True)

            @pl.when(j + 4 < NBW)
            def _():
                start_idx(j + 4, u, sis[u])

            uc = (u + 2) % 4
            wait_idx(uc, sis[uc])
            pltpu.async_copy(pre_hbm.at[idx_s.at[uc]], r, sg)

        def body(k4, carry):
            j = k4 * 4
            sub(j, 0)
            sub(j + 1, 1)
            sub(j + 2, 2)
            sub(j + 3, 3)
            return carry

        lax.fori_loop(0, NBW // 4, body, 0)
        for u in (0, 1):
            r = rowbufs[u % 2]
            sg = sgs[u % 2]
            pltpu.make_async_copy(pre_hbm.at[idx_s.at[u]], r, sg).wait()
            pltpu.sync_copy(r, acc.at[idx_d.at[u]], add=True)

        plsc.subcore_barrier()

        @pl.when(c == 0)
        def _():
            pltpu.sync_copy(acc.at[pl.ds(row0, RPT)], out0_hbm.at[pl.ds(row0, RPT)])

            @pl.when(s == NS - 1)
            def _():
                pltpu.sync_copy(acc.at[pl.ds(TAIL0, TAIL)],
                                out0_hbm.at[pl.ds(TAIL0, TAIL)])

        @pl.when(c == 1)
        def _():
            pltpu.sync_copy(acc.at[pl.ds(row0, RPT)], out1_hbm.at[pl.ds(row0, RPT)])

            @pl.when(s == NS - 1)
            def _():
                pltpu.sync_copy(acc.at[pl.ds(TAIL0, TAIL)],
                                out1_hbm.at[pl.ds(TAIL0, TAIL)])

    return k(pre, srcc, dstc, zeros)


def kernel(x, edge_index0, edge_index1, W0, W1, b):
    src0 = edge_index0[0].astype(jnp.int32)
    dst0 = edge_index0[1].astype(jnp.int32)
    src1 = edge_index1[0].astype(jnp.int32)
    dst1 = edge_index1[1].astype(jnp.int32)
    srcc = jnp.concatenate([src0, src1 + N])
    dstc = jnp.concatenate([dst0, dst1])
    wcat = jnp.stack([W0, W1])
    pre = pl.pallas_call(
        _mm_body,
        grid=(10,),
        in_specs=[pl.BlockSpec((2000, D), lambda i: (i % 5, 0)),
                  pl.BlockSpec((1, D, D), lambda i: (i // 5, 0, 0))],
        out_specs=pl.BlockSpec((2000, D), lambda i: (i, 0)),
        out_shape=jax.ShapeDtypeStruct((2 * N, D), jnp.float32),
    )(x, wcat)
    zeros = jnp.zeros((N, D), jnp.float32)
    part0, part1 = _sc_segment_sum(pre, srcc, dstc, zeros)
    b2 = jnp.reshape(b, (1, D))
    out = pl.pallas_call(
        _combine_body,
        grid=(5,),
        in_specs=[pl.BlockSpec((2000, D), lambda i: (i, 0)),
                  pl.BlockSpec((2000, D), lambda i: (i, 0)),
                  pl.BlockSpec((1, D), lambda i: (0, 0))],
        out_specs=pl.BlockSpec((2000, D), lambda i: (i, 0)),
        out_shape=jax.ShapeDtypeStruct((N, D), jnp.float32),
    )(part0, part1, b2)
    return out


# speedup vs baseline: 11.2505x; 1.0266x over previous
import functools

import jax
import jax.numpy as jnp
from jax import lax
from jax.experimental import pallas as pl
from jax.experimental.pallas import tpu as pltpu
from jax.experimental.pallas import tpu_sc as plsc

N = 10000
D = 128
E = 320000
NC = 2
NS = 16
NW = NC * NS
EPW = 2 * E // NW
BLK = 80
NBW = EPW // BLK
RPT = 624
TAIL = N - RPT * NS
TAIL0 = RPT * NS


def _mm_body(x_ref, w_ref, o_ref):
    o_ref[...] = jnp.dot(x_ref[...], w_ref[0],
                         preferred_element_type=jnp.float32)


def _combine_body(p0_ref, p1_ref, b_ref, o_ref):
    o_ref[...] = jnp.maximum(p0_ref[...] + p1_ref[...] + b_ref[...], 0.0)


def _sc_segment_sum(pre, srcc, dstc, zeros):
    mesh = plsc.VectorSubcoreMesh(core_axis_name="c", subcore_axis_name="s")

    @functools.partial(
        pl.kernel,
        mesh=mesh,
        out_type=[jax.ShapeDtypeStruct((N, D), jnp.float32)] * 2,
        scratch_types=[
            pltpu.VMEM_SHARED((N, D), jnp.float32),
            pltpu.VMEM((8, BLK), jnp.int32),
            pltpu.VMEM((8, BLK), jnp.int32),
            pltpu.VMEM((BLK, D), jnp.float32),
            pltpu.VMEM((BLK, D), jnp.float32),
            pltpu.VMEM((BLK, D), jnp.float32),
            pltpu.VMEM((BLK, D), jnp.float32),
        ] + [pltpu.SemaphoreType.DMA] * 16,
    )
    def k(pre_hbm, src_hbm, dst_hbm, z_hbm, out0_hbm, out1_hbm,
          acc, idx_s, idx_d, r0, r1, r2, r3, *sems):
        c = lax.axis_index("c")
        s = lax.axis_index("s")
        wid = c * NS + s
        row0 = s * RPT
        base = wid * EPW
        sis = sems[0:8]
        sgs = sems[8:12]
        sss = sems[12:16]
        rows = (r0, r1, r2, r3)

        def start_idx(j, v):
            pltpu.async_copy(src_hbm.at[pl.ds(base + j * BLK, BLK)],
                             idx_s.at[v], sis[v])
            pltpu.async_copy(dst_hbm.at[pl.ds(base + j * BLK, BLK)],
                             idx_d.at[v], sis[v])

        def wait_idx(v):
            pltpu.make_async_copy(src_hbm.at[pl.ds(0, BLK)],
                                  idx_s.at[v], sis[v]).wait()
            pltpu.make_async_copy(dst_hbm.at[pl.ds(0, BLK)],
                                  idx_d.at[v], sis[v]).wait()

        def start_gather(u, v):
            pltpu.async_copy(pre_hbm.at[idx_s.at[v]], rows[u], sgs[u])

        def wait_gather(u, v):
            pltpu.make_async_copy(pre_hbm.at[idx_s.at[v]], rows[u],
                                  sgs[u]).wait()

        def wait_scatter(u, v):
            pltpu.make_async_copy(rows[u], acc.at[idx_d.at[v]],
                                  sss[u]).wait()

        pltpu.sync_copy(z_hbm.at[pl.ds(row0, RPT)], acc.at[pl.ds(row0, RPT)])

        @pl.when(s == NS - 1)
        def _():
            pltpu.sync_copy(z_hbm.at[pl.ds(TAIL0, TAIL)],
                            acc.at[pl.ds(TAIL0, TAIL)])

        plsc.subcore_barrier()

        def sub(j, t, static_edge):
            u, v = t % 4, t % 8
            uc2, vg, vp = (t + 2) % 4, (t + 2) % 8, (t + 6) % 8
            wait_gather(u, v)
            pltpu.async_copy(rows[u], acc.at[idx_d.at[v]], sss[u], add=True)
            if static_edge:
                if t >= 2:
                    wait_scatter(uc2, vp)
                start_idx(j + 6, vp)
            else:
                wait_scatter(uc2, vp)

                @pl.when(j + 6 < NBW)
                def _():
                    start_idx(j + 6, vp)

            wait_idx(vg)
            start_gather(uc2, vg)

        for v in range(6):
            start_idx(v, v)
        wait_idx(0)
        start_gather(0, 0)
        wait_idx(1)
        start_gather(1, 1)
        for j in range(8):
            sub(j, j, True)

        def body(k8, carry):
            j8 = 8 + k8 * 8
            for t in range(8):
                sub(j8 + t, t, False)
            return carry

        lax.fori_loop(0, (NBW - 8) // 8, body, 0)
        for u in (0, 1):
            wait_gather(u, u)
            pltpu.sync_copy(rows[u], acc.at[idx_d.at[u]], add=True)
        wait_scatter(2, 6)
        wait_scatter(3, 7)

        plsc.subcore_barrier()

        @pl.when(c == 0)
        def _():
            pltpu.sync_copy(acc.at[pl.ds(row0, RPT)], out0_hbm.at[pl.ds(row0, RPT)])

            @pl.when(s == NS - 1)
            def _():
                pltpu.sync_copy(acc.at[pl.ds(TAIL0, TAIL)],
                                out0_hbm.at[pl.ds(TAIL0, TAIL)])

        @pl.when(c == 1)
        def _():
            pltpu.sync_copy(acc.at[pl.ds(row0, RPT)], out1_hbm.at[pl.ds(row0, RPT)])

            @pl.when(s == NS - 1)
            def _():
                pltpu.sync_copy(acc.at[pl.ds(TAIL0, TAIL)],
                                out1_hbm.at[pl.ds(TAIL0, TAIL)])

    return k(pre, srcc, dstc, zeros)


def kernel(x, edge_index0, edge_index1, W0, W1, b):
    src0 = edge_index0[0].astype(jnp.int32)
    dst0 = edge_index0[1].astype(jnp.int32)
    src1 = edge_index1[0].astype(jnp.int32)
    dst1 = edge_index1[1].astype(jnp.int32)
    srcc = jnp.concatenate([src0, src1 + N])
    dstc = jnp.concatenate([dst0, dst1])
    wcat = jnp.stack([W0, W1])
    pre = pl.pallas_call(
        _mm_body,
        grid=(10,),
        in_specs=[pl.BlockSpec((2000, D), lambda i: (i % 5, 0)),
                  pl.BlockSpec((1, D, D), lambda i: (i // 5, 0, 0))],
        out_specs=pl.BlockSpec((2000, D), lambda i: (i, 0)),
        out_shape=jax.ShapeDtypeStruct((2 * N, D), jnp.float32),
    )(x, wcat)
    zeros = jnp.zeros((N, D), jnp.float32)
    part0, part1 = _sc_segment_sum(pre, srcc, dstc, zeros)
    b2 = jnp.reshape(b, (1, D))
    out = pl.pallas_call(
        _combine_body,
        grid=(5,),
        in_specs=[pl.BlockSpec((2000, D), lambda i: (i, 0)),
                  pl.BlockSpec((2000, D), lambda i: (i, 0)),
                  pl.BlockSpec((1, D), lambda i: (0, 0))],
        out_specs=pl.BlockSpec((2000, D), lambda i: (i, 0)),
        out_shape=jax.ShapeDtypeStruct((N, D), jnp.float32),
    )(part0, part1, b2)
    return out


# speedup vs baseline: 12.4695x; 1.1084x over previous
import functools

import jax
import jax.numpy as jnp
from jax import lax
from jax.experimental import pallas as pl
from jax.experimental.pallas import tpu as pltpu
from jax.experimental.pallas import tpu_sc as plsc

N = 10000
D = 128
E = 320000
NC = 2
NS = 16
NW = NC * NS
BLK = 120
NBW = 166
TBLK = 80
EPW = NBW * BLK
TBASE = EPW * NW
RPT = 624
TAIL = N - RPT * NS
TAIL0 = RPT * NS


def _mm_body(x_ref, w_ref, o_ref):
    o_ref[...] = jnp.dot(x_ref[...], w_ref[0],
                         preferred_element_type=jnp.float32)


def _combine_body(p0_ref, p1_ref, b_ref, o_ref):
    o_ref[...] = jnp.maximum(p0_ref[...] + p1_ref[...] + b_ref[...], 0.0)


def _sc_segment_sum(pre, srcc, dstc, zeros):
    mesh = plsc.VectorSubcoreMesh(core_axis_name="c", subcore_axis_name="s")

    @functools.partial(
        pl.kernel,
        mesh=mesh,
        out_type=[jax.ShapeDtypeStruct((N, D), jnp.float32)] * 2,
        scratch_types=[
            pltpu.VMEM_SHARED((N, D), jnp.float32),
            pltpu.VMEM((6, BLK), jnp.int32),
            pltpu.VMEM((6, BLK), jnp.int32),
            pltpu.VMEM((BLK, D), jnp.float32),
            pltpu.VMEM((BLK, D), jnp.float32),
            pltpu.VMEM((BLK, D), jnp.float32),
            pltpu.VMEM((TBLK,), jnp.int32),
            pltpu.VMEM((TBLK,), jnp.int32),
        ] + [pltpu.SemaphoreType.DMA] * 13,
    )
    def k(pre_hbm, src_hbm, dst_hbm, z_hbm, out0_hbm, out1_hbm,
          acc, idx_s, idx_d, r0, r1, r2, tis, tid, *sems):
        c = lax.axis_index("c")
        s = lax.axis_index("s")
        wid = c * NS + s
        row0 = s * RPT
        base = wid * EPW
        sis = sems[0:6]
        sgs = sems[6:9]
        sss = sems[9:12]
        sts = sems[12]
        rows = (r0, r1, r2)

        def start_idx(j, v):
            pltpu.async_copy(src_hbm.at[pl.ds(base + j * BLK, BLK)],
                             idx_s.at[v], sis[v])
            pltpu.async_copy(dst_hbm.at[pl.ds(base + j * BLK, BLK)],
                             idx_d.at[v], sis[v])

        def wait_idx(v):
            pltpu.make_async_copy(src_hbm.at[pl.ds(0, BLK)],
                                  idx_s.at[v], sis[v]).wait()
            pltpu.make_async_copy(dst_hbm.at[pl.ds(0, BLK)],
                                  idx_d.at[v], sis[v]).wait()

        def start_gather(u, v):
            pltpu.async_copy(pre_hbm.at[idx_s.at[v]], rows[u], sgs[u])

        def wait_gather(u, v):
            pltpu.make_async_copy(pre_hbm.at[idx_s.at[v]], rows[u],
                                  sgs[u]).wait()

        def wait_scatter(u, v):
            pltpu.make_async_copy(rows[u], acc.at[idx_d.at[v]],
                                  sss[u]).wait()

        pltpu.async_copy(src_hbm.at[pl.ds(TBASE + wid * TBLK, TBLK)], tis, sts)
        pltpu.async_copy(dst_hbm.at[pl.ds(TBASE + wid * TBLK, TBLK)], tid, sts)

        pltpu.sync_copy(z_hbm.at[pl.ds(row0, RPT)], acc.at[pl.ds(row0, RPT)])

        @pl.when(s == NS - 1)
        def _():
            pltpu.sync_copy(z_hbm.at[pl.ds(TAIL0, TAIL)],
                            acc.at[pl.ds(TAIL0, TAIL)])

        plsc.subcore_barrier()

        def sub(j, t, first, do_idx, do_gather):
            u, v = t % 3, t % 6
            un = up = (t + 2) % 3
            vg, vp = (t + 2) % 6, (t + 4) % 6
            wait_gather(u, v)
            pltpu.async_copy(rows[u], acc.at[idx_d.at[v]], sss[u], add=True)
            if not first:
                wait_scatter(up, vp)
            if do_idx:
                start_idx(j + 4, vp)
            if do_gather:
                wait_idx(vg)
                start_gather(un, vg)

        for v in range(4):
            start_idx(v, v)
        wait_idx(0)
        start_gather(0, 0)
        wait_idx(1)
        start_gather(1, 1)
        for j in range(6):
            sub(j, j, j == 0, True, True)

        def body(k6, carry):
            j6 = 6 + k6 * 6
            for t in range(6):
                sub(j6 + t, t, False, True, True)
            return carry

        lax.fori_loop(0, (NBW - 10) // 6, body, 0)
        jp = NBW - 4
        for i in range(4):
            j = jp + i
            sub(j, j % 6, False, False, i < 2)
        wait_scatter((NBW - 1) % 3, (NBW - 1) % 6)
        pltpu.make_async_copy(src_hbm.at[pl.ds(0, TBLK)], tis, sts).wait()
        pltpu.make_async_copy(dst_hbm.at[pl.ds(0, TBLK)], tid, sts).wait()
        pltpu.async_copy(pre_hbm.at[tis], r2.at[pl.ds(0, TBLK)], sts).wait()
        pltpu.sync_copy(r2.at[pl.ds(0, TBLK)], acc.at[tid], add=True)

        plsc.subcore_barrier()

        @pl.when(c == 0)
        def _():
            pltpu.sync_copy(acc.at[pl.ds(row0, RPT)], out0_hbm.at[pl.ds(row0, RPT)])

            @pl.when(s == NS - 1)
            def _():
                pltpu.sync_copy(acc.at[pl.ds(TAIL0, TAIL)],
                                out0_hbm.at[pl.ds(TAIL0, TAIL)])

        @pl.when(c == 1)
        def _():
            pltpu.sync_copy(acc.at[pl.ds(row0, RPT)], out1_hbm.at[pl.ds(row0, RPT)])

            @pl.when(s == NS - 1)
            def _():
                pltpu.sync_copy(acc.at[pl.ds(TAIL0, TAIL)],
                                out1_hbm.at[pl.ds(TAIL0, TAIL)])

    return k(pre, srcc, dstc, zeros)


def kernel(x, edge_index0, edge_index1, W0, W1, b):
    src0 = edge_index0[0].astype(jnp.int32)
    dst0 = edge_index0[1].astype(jnp.int32)
    src1 = edge_index1[0].astype(jnp.int32)
    dst1 = edge_index1[1].astype(jnp.int32)
    srcc = jnp.concatenate([src0, src1 + N])
    dstc = jnp.concatenate([dst0, dst1])
    wcat = jnp.stack([W0, W1])
    pre = pl.pallas_call(
        _mm_body,
        grid=(10,),
        in_specs=[pl.BlockSpec((2000, D), lambda i: (i % 5, 0)),
                  pl.BlockSpec((1, D, D), lambda i: (i // 5, 0, 0))],
        out_specs=pl.BlockSpec((2000, D), lambda i: (i, 0)),
        out_shape=jax.ShapeDtypeStruct((2 * N, D), jnp.float32),
    )(x, wcat)
    zeros = jnp.zeros((N, D), jnp.float32)
    part0, part1 = _sc_segment_sum(pre, srcc, dstc, zeros)
    b2 = jnp.reshape(b, (1, D))
    out = pl.pallas_call(
        _combine_body,
        grid=(5,),
        in_specs=[pl.BlockSpec((2000, D), lambda i: (i, 0)),
                  pl.BlockSpec((2000, D), lambda i: (i, 0)),
                  pl.BlockSpec((1, D), lambda i: (0, 0))],
        out_specs=pl.BlockSpec((2000, D), lambda i: (i, 0)),
        out_shape=jax.ShapeDtypeStruct((N, D), jnp.float32),
    )(part0, part1, b2)
    return out


# speedup vs baseline: 12.5593x; 1.0072x over previous
import functools

import jax
import jax.numpy as jnp
from jax import lax
from jax.experimental import pallas as pl
from jax.experimental.pallas import tpu as pltpu
from jax.experimental.pallas import tpu_sc as plsc

N = 10000
D = 128
E = 320000
NC = 2
NS = 16
NW = NC * NS
BLK = 120
NBW = 166
TBLK = 80
EPW = NBW * BLK
TBASE = EPW * NW
RPT = 624
TAIL = N - RPT * NS
TAIL0 = RPT * NS


def _mm_body(x_ref, w_ref, o_ref):
    o_ref[...] = jnp.dot(x_ref[...], w_ref[0],
                         preferred_element_type=jnp.float32)


def _combine_body(p0_ref, p1_ref, b_ref, o_ref):
    o_ref[...] = jnp.maximum(p0_ref[...] + p1_ref[...] + b_ref[...], 0.0)


def _sc_segment_sum(pre, srcc, dstc, zeros):
    mesh = plsc.VectorSubcoreMesh(core_axis_name="c", subcore_axis_name="s")

    @functools.partial(
        pl.kernel,
        mesh=mesh,
        out_type=[jax.ShapeDtypeStruct((N, D), jnp.float32)] * 2,
        scratch_types=[
            pltpu.VMEM_SHARED((N, D), jnp.float32),
            pltpu.VMEM((6, BLK), jnp.int32),
            pltpu.VMEM((6, BLK), jnp.int32),
            pltpu.VMEM((BLK, D), jnp.float32),
            pltpu.VMEM((BLK, D), jnp.float32),
            pltpu.VMEM((BLK, D), jnp.float32),
            pltpu.VMEM((TBLK,), jnp.int32),
            pltpu.VMEM((TBLK,), jnp.int32),
        ] + [pltpu.SemaphoreType.DMA] * 13,
    )
    def k(pre_hbm, src_hbm, dst_hbm, z_hbm, out0_hbm, out1_hbm,
          acc, idx_s, idx_d, r0, r1, r2, tis, tid, *sems):
        c = lax.axis_index("c")
        s = lax.axis_index("s")
        wid = c * NS + s
        row0 = s * RPT
        base = wid * EPW
        sis = sems[0:6]
        sgs = sems[6:9]
        sss = sems[9:12]
        sts = sems[12]
        rows = (r0, r1, r2)

        def start_idx(j, v):
            pltpu.async_copy(src_hbm.at[pl.ds(base + j * BLK, BLK)],
                             idx_s.at[v], sis[v])
            pltpu.async_copy(dst_hbm.at[pl.ds(base + j * BLK, BLK)],
                             idx_d.at[v], sis[v])

        def wait_idx(v):
            pltpu.make_async_copy(src_hbm.at[pl.ds(0, BLK)],
                                  idx_s.at[v], sis[v]).wait()
            pltpu.make_async_copy(dst_hbm.at[pl.ds(0, BLK)],
                                  idx_d.at[v], sis[v]).wait()

        def start_gather(u, v):
            pltpu.async_copy(pre_hbm.at[idx_s.at[v]], rows[u], sgs[u])

        def wait_gather(u, v):
            pltpu.make_async_copy(pre_hbm.at[idx_s.at[v]], rows[u],
                                  sgs[u]).wait()

        def wait_scatter(u, v):
            pltpu.make_async_copy(rows[u], acc.at[idx_d.at[v]],
                                  sss[u]).wait()

        pltpu.async_copy(src_hbm.at[pl.ds(TBASE + wid * TBLK, TBLK)], tis, sts)
        pltpu.async_copy(dst_hbm.at[pl.ds(TBASE + wid * TBLK, TBLK)], tid, sts)
        for v in range(4):
            start_idx(v, v)
        wait_idx(0)
        start_gather(0, 0)
        wait_idx(1)
        start_gather(1, 1)

        pltpu.sync_copy(z_hbm.at[pl.ds(row0, RPT)], acc.at[pl.ds(row0, RPT)])

        @pl.when(s == NS - 1)
        def _():
            pltpu.sync_copy(z_hbm.at[pl.ds(TAIL0, TAIL)],
                            acc.at[pl.ds(TAIL0, TAIL)])

        plsc.subcore_barrier()

        def sub(j, t, first, do_idx, do_gather):
            u, v = t % 3, t % 6
            un = up = (t + 2) % 3
            vg, vp = (t + 2) % 6, (t + 4) % 6
            wait_gather(u, v)
            pltpu.async_copy(rows[u], acc.at[idx_d.at[v]], sss[u], add=True)
            if not first:
                wait_scatter(up, vp)
            if do_idx:
                start_idx(j + 4, vp)
            if do_gather:
                wait_idx(vg)
                start_gather(un, vg)

        for j in range(6):
            sub(j, j, j == 0, True, True)

        def body(k6, carry):
            j6 = 6 + k6 * 6
            for t in range(6):
                sub(j6 + t, t, False, True, True)
            return carry

        lax.fori_loop(0, (NBW - 10) // 6, body, 0)
        jp = NBW - 4
        for i in range(4):
            j = jp + i
            sub(j, j % 6, False, False, i < 2)
        wait_scatter((NBW - 1) % 3, (NBW - 1) % 6)
        pltpu.make_async_copy(src_hbm.at[pl.ds(0, TBLK)], tis, sts).wait()
        pltpu.make_async_copy(dst_hbm.at[pl.ds(0, TBLK)], tid, sts).wait()
        pltpu.async_copy(pre_hbm.at[tis], r2.at[pl.ds(0, TBLK)], sts).wait()
        pltpu.sync_copy(r2.at[pl.ds(0, TBLK)], acc.at[tid], add=True)

        plsc.subcore_barrier()

        @pl.when(c == 0)
        def _():
            pltpu.sync_copy(acc.at[pl.ds(row0, RPT)], out0_hbm.at[pl.ds(row0, RPT)])

            @pl.when(s == NS - 1)
            def _():
                pltpu.sync_copy(acc.at[pl.ds(TAIL0, TAIL)],
                                out0_hbm.at[pl.ds(TAIL0, TAIL)])

        @pl.when(c == 1)
        def _():
            pltpu.sync_copy(acc.at[pl.ds(row0, RPT)], out1_hbm.at[pl.ds(row0, RPT)])

            @pl.when(s == NS - 1)
            def _():
                pltpu.sync_copy(acc.at[pl.ds(TAIL0, TAIL)],
                                out1_hbm.at[pl.ds(TAIL0, TAIL)])

    return k(pre, srcc, dstc, zeros)


def kernel(x, edge_index0, edge_index1, W0, W1, b):
    src0 = edge_index0[0].astype(jnp.int32)
    dst0 = edge_index0[1].astype(jnp.int32)
    src1 = edge_index1[0].astype(jnp.int32)
    dst1 = edge_index1[1].astype(jnp.int32)
    srcc = jnp.concatenate([src0, src1 + N])
    dstc = jnp.concatenate([dst0, dst1])
    wcat = jnp.stack([W0, W1])
    pre = pl.pallas_call(
        _mm_body,
        grid=(10,),
        in_specs=[pl.BlockSpec((2000, D), lambda i: (i % 5, 0)),
                  pl.BlockSpec((1, D, D), lambda i: (i // 5, 0, 0))],
        out_specs=pl.BlockSpec((2000, D), lambda i: (i, 0)),
        out_shape=jax.ShapeDtypeStruct((2 * N, D), jnp.float32),
    )(x, wcat)
    zeros = jnp.zeros((N, D), jnp.float32)
    part0, part1 = _sc_segment_sum(pre, srcc, dstc, zeros)
    b2 = jnp.reshape(b, (1, D))
    out = pl.pallas_call(
        _combine_body,
        grid=(5,),
        in_specs=[pl.BlockSpec((2000, D), lambda i: (i, 0)),
                  pl.BlockSpec((2000, D), lambda i: (i, 0)),
                  pl.BlockSpec((1, D), lambda i: (0, 0))],
        out_specs=pl.BlockSpec((2000, D), lambda i: (i, 0)),
        out_shape=jax.ShapeDtypeStruct((N, D), jnp.float32),
    )(part0, part1, b2)
    return out
